# baseline (speedup 1.0000x reference)
import functools

import jax
import jax.numpy as jnp
from jax import lax
from jax.experimental import pallas as pl
from jax.experimental.pallas import tpu as pltpu
from jax.experimental.pallas import tpu_sc as plsc

B = 1024
L = 512
E = 128
VOCAB = 100000

NC = 2
NS = 16
NW = NC * NS
BPW = B // NW
CL = 128
NLC = L // CL
LANES = 16
EV = E // LANES


def _emb_body(seq_hbm, seg_hbm, tok_hbm, pos_hbm, segtab_hbm, out_hbm,
              idx0, idx1, sg0, sg1, rows0, rows1, pos_v, posseg_v, segtab_v,
              gsem0, gsem1):
    cid = lax.axis_index("c")
    sid = lax.axis_index("s")
    wid = sid * NC + cid
    wbase = wid * BPW * L

    idx = (idx0, idx1)
    sg = (sg0, sg1)
    rows = (rows0, rows1)
    gsem = (gsem0, gsem1)

    pltpu.sync_copy(segtab_hbm, segtab_v)
    segrows = [[segtab_v[s, pl.ds(j * LANES, LANES)] for j in range(EV)]
               for s in range(3)]

    for lc in range(NLC):
        pltpu.sync_copy(pos_hbm.at[pl.ds(lc * CL, CL)], pos_v)

        def pos_body(i, _):
            for j in range(EV):
                sl = pl.ds(j * LANES, LANES)
                p = pos_v[i, sl]
                for s in range(3):
                    posseg_v[s * CL + i, sl] = p + segrows[s][j]
            return 0

        lax.fori_loop(0, CL, pos_body, 0)

        def fetch_and_gather(c, k, lc=lc):
            base = pl.multiple_of(wbase + lc * CL + c * L, CL)
            pltpu.sync_copy(seq_hbm.at[pl.ds(base, CL)], idx[k])
            pltpu.sync_copy(seg_hbm.at[pl.ds(base, CL)], sg[k])
            pltpu.async_copy(tok_hbm.at[idx[k]], rows[k], gsem[k])

        def compute_and_write(c, k, lc=lc):
            base = pl.multiple_of(wbase + lc * CL + c * L, CL)
            pltpu.make_async_copy(tok_hbm.at[idx[k]], rows[k],
                                  gsem[k]).wait()

            def group_body(g, _):
                i0 = pl.multiple_of(g * LANES, LANES)
                s16 = sg[k][pl.ds(i0, LANES)]
                prow = s16 * CL
                for kk in range(LANES):
                    r = i0 + kk
                    pr = prow[kk] + r
                    sls = [pl.ds(j * LANES, LANES) for j in range(EV)]
                    ts = [rows[k][r, sl] for sl in sls]
                    ps = [posseg_v[pr, sl] for sl in sls]
                    for j in range(EV):
                        rows[k][r, sls[j]] = ts[j] + ps[j]
                return 0

            lax.fori_loop(0, CL // LANES, group_body, 0)
            pltpu.sync_copy(rows[k], out_hbm.at[pl.ds(base, CL)])

        fetch_and_gather(0, 0)

        def pair_body(t, _):
            for k in (0, 1):
                c = t * 2 + k

                @pl.when(c + 1 < BPW)
                def _():
                    fetch_and_gather(c + 1, k ^ 1)

                compute_and_write(c, k)
            return 0

        lax.fori_loop(0, BPW // 2, pair_body, 0)


@functools.partial(jax.jit, static_argnames=())
def kernel(sequence, segment_label, token_table, position_table,
           segment_table):
    seq = sequence.reshape(-1).astype(jnp.int32)
    seg = segment_label.reshape(-1).astype(jnp.int32)

    mesh = plsc.VectorSubcoreMesh(core_axis_name="c", subcore_axis_name="s",
                                  num_cores=NC, num_subcores=NS)
    out = pl.kernel(
        _emb_body,
        out_type=jax.ShapeDtypeStruct((B * L, E), jnp.float32),
        mesh=mesh,
        scratch_types=[
            pltpu.VMEM((CL,), jnp.int32),
            pltpu.VMEM((CL,), jnp.int32),
            pltpu.VMEM((CL,), jnp.int32),
            pltpu.VMEM((CL,), jnp.int32),
            pltpu.VMEM((CL, E), jnp.float32),
            pltpu.VMEM((CL, E), jnp.float32),
            pltpu.VMEM((CL, E), jnp.float32),
            pltpu.VMEM((3 * CL, E), jnp.float32),
            pltpu.VMEM((3, E), jnp.float32),
            pltpu.SemaphoreType.DMA,
            pltpu.SemaphoreType.DMA,
        ],
    )(seq, seg, token_table, position_table, segment_table)
    return out.reshape(B, L, E)


# speedup vs baseline: 1.1004x; 1.1004x over previous
import functools

import jax
import jax.numpy as jnp
from jax import lax
from jax.experimental import pallas as pl
from jax.experimental.pallas import tpu as pltpu
from jax.experimental.pallas import tpu_sc as plsc

B = 1024
L = 512
E = 128
VOCAB = 100000

NC = 2
NS = 16
NW = NC * NS
BPW = B // NW
CL = 128
NLC = L // CL
LANES = 16
EV = E // LANES


def _emb_body(seq_hbm, seg_hbm, tok_hbm, pos_hbm, segtab_hbm, out_hbm,
              idx0, idx1, sg0, sg1, rows0, rows1, wb0, wb1,
              posseg_v, segtab_v,
              gsem0, gsem1, osem0, osem1):
    cid = lax.axis_index("c")
    sid = lax.axis_index("s")
    wid = sid * NC + cid
    wbase = wid * BPW * L

    idx = (idx0, idx1)
    sg = (sg0, sg1)
    rows = (rows0, rows1)
    wb = (wb0, wb1)
    gsem = (gsem0, gsem1)
    osem = (osem0, osem1)

    pltpu.sync_copy(segtab_hbm, segtab_v)
    segrows = [[segtab_v[s, pl.ds(j * LANES, LANES)] for j in range(EV)]
               for s in range(3)]

    for lc in range(NLC):
        for s in range(3):
            pltpu.sync_copy(pos_hbm.at[pl.ds(lc * CL, CL)],
                            posseg_v.at[pl.ds(s * CL, CL)])

        def pos_body(i, _):
            for s in range(3):
                for j in range(EV):
                    sl = pl.ds(j * LANES, LANES)
                    r = s * CL + i
                    posseg_v[r, sl] = posseg_v[r, sl] + segrows[s][j]
            return 0

        lax.fori_loop(0, CL, pos_body, 0)

        def fetch_and_gather(c, k, lc=lc):
            base = pl.multiple_of(wbase + lc * CL + c * L, CL)
            pltpu.sync_copy(seq_hbm.at[pl.ds(base, CL)], idx[k])
            pltpu.sync_copy(seg_hbm.at[pl.ds(base, CL)], sg[k])
            pltpu.async_copy(tok_hbm.at[idx[k]], rows[k], gsem[k])

        def compute_and_write(c, k, lc=lc):
            base = pl.multiple_of(wbase + lc * CL + c * L, CL)
            pltpu.make_async_copy(tok_hbm.at[idx[k]], rows[k],
                                  gsem[k]).wait()

            @pl.when(c >= 2)
            def _():
                pb = pl.multiple_of(base - 2 * L, CL)
                pltpu.make_async_copy(wb[k], out_hbm.at[pl.ds(pb, CL)],
                                      osem[k]).wait()

            def group_body(g, _):
                i0 = pl.multiple_of(g * LANES, LANES)
                s16 = sg[k][pl.ds(i0, LANES)]
                prow = s16 * CL
                for kk in range(LANES):
                    r = i0 + kk
                    pr = prow[kk] + r
                    sls = [pl.ds(j * LANES, LANES) for j in range(EV)]
                    ts = [rows[k][r, sl] for sl in sls]
                    ps = [posseg_v[pr, sl] for sl in sls]
                    for j in range(EV):
                        wb[k][r, sls[j]] = ts[j] + ps[j]
                return 0

            lax.fori_loop(0, CL // LANES, group_body, 0)
            pltpu.async_copy(wb[k], out_hbm.at[pl.ds(base, CL)], osem[k])

        fetch_and_gather(0, 0)

        def pair_body(t, _):
            for k in (0, 1):
                c = t * 2 + k

                @pl.when(c + 1 < BPW)
                def _():
                    fetch_and_gather(c + 1, k ^ 1)

                compute_and_write(c, k)
            return 0

        lax.fori_loop(0, BPW // 2, pair_body, 0)

        for c in (BPW - 2, BPW - 1):
            k = c % 2
            base = pl.multiple_of(wbase + lc * CL + c * L, CL)
            pltpu.make_async_copy(wb[k], out_hbm.at[pl.ds(base, CL)],
                                  osem[k]).wait()


@functools.partial(jax.jit, static_argnames=())
def kernel(sequence, segment_label, token_table, position_table,
           segment_table):
    seq = sequence.reshape(-1).astype(jnp.int32)
    seg = segment_label.reshape(-1).astype(jnp.int32)

    mesh = plsc.VectorSubcoreMesh(core_axis_name="c", subcore_axis_name="s",
                                  num_cores=NC, num_subcores=NS)
    out = pl.kernel(
        _emb_body,
        out_type=jax.ShapeDtypeStruct((B * L, E), jnp.float32),
        mesh=mesh,
        scratch_types=[
            pltpu.VMEM((CL,), jnp.int32),
            pltpu.VMEM((CL,), jnp.int32),
            pltpu.VMEM((CL,), jnp.int32),
            pltpu.VMEM((CL,), jnp.int32),
            pltpu.VMEM((CL, E), jnp.float32),
            pltpu.VMEM((CL, E), jnp.float32),
            pltpu.VMEM((CL, E), jnp.float32),
            pltpu.VMEM((CL, E), jnp.float32),
            pltpu.VMEM((3 * CL, E), jnp.float32),
            pltpu.VMEM((3, E), jnp.float32),
            pltpu.SemaphoreType.DMA,
            pltpu.SemaphoreType.DMA,
            pltpu.SemaphoreType.DMA,
            pltpu.SemaphoreType.DMA,
        ],
    )(seq, seg, token_table, position_table, segment_table)
    return out.reshape(B, L, E)


# speedup vs baseline: 1.5464x; 1.4053x over previous
import functools

import jax
import jax.numpy as jnp
from jax import lax
from jax.experimental import pallas as pl
from jax.experimental.pallas import tpu as pltpu
from jax.experimental.pallas import tpu_sc as plsc

B = 1024
L = 512
E = 128
VOCAB = 100000

NC = 2
NS = 16
NW = NC * NS
BPW = B // NW
CL = 128
NLC = L // CL
LANES = 16
EV = E // LANES


def _emb_body(seq_hbm, seg_hbm, tok_hbm, pos_hbm, segtab_hbm, out_hbm,
              idx_all, seg_all, rows0, rows1, wb0, wb1,
              posseg_v, segtab_v,
              gsem0, gsem1, osem0, osem1):
    cid = lax.axis_index("c")
    sid = lax.axis_index("s")
    wid = sid * NC + cid
    wbase = wid * BPW * L

    rows = (rows0, rows1)
    wb = (wb0, wb1)
    gsem = (gsem0, gsem1)
    osem = (osem0, osem1)

    pltpu.sync_copy(segtab_hbm, segtab_v)
    segrows = [[segtab_v[s, pl.ds(j * LANES, LANES)] for j in range(EV)]
               for s in range(3)]

    for lc in range(NLC):
        pltpu.sync_copy(
            seq_hbm.at[pl.ds(wid * BPW, BPW), pl.ds(lc * CL, CL)], idx_all)
        pltpu.sync_copy(
            seg_hbm.at[pl.ds(wid * BPW, BPW), pl.ds(lc * CL, CL)], seg_all)

        for s in range(3):
            pltpu.sync_copy(pos_hbm.at[pl.ds(lc * CL, CL)],
                            posseg_v.at[pl.ds(s * CL, CL)])

        def pos_body(i, _):
            for s in range(3):
                for j in range(EV):
                    sl = pl.ds(j * LANES, LANES)
                    r = s * CL + i
                    posseg_v[r, sl] = posseg_v[r, sl] + segrows[s][j]
            return 0

        lax.fori_loop(0, CL, pos_body, 0)

        def fetch_and_gather(c, k, lc=lc):
            pltpu.async_copy(tok_hbm.at[idx_all.at[c]], rows[k], gsem[k])

        def compute_and_write(c, k, lc=lc):
            base = pl.multiple_of(wbase + lc * CL + c * L, CL)
            pltpu.make_async_copy(tok_hbm.at[idx_all.at[c]], rows[k],
                                  gsem[k]).wait()

            @pl.when(c >= 2)
            def _():
                pb = pl.multiple_of(base - 2 * L, CL)
                pltpu.make_async_copy(wb[k], out_hbm.at[pl.ds(pb, CL)],
                                      osem[k]).wait()

            def group_body(g, _):
                i0 = pl.multiple_of(g * LANES, LANES)
                s16 = seg_all[c, pl.ds(i0, LANES)]
                prow = s16 * CL
                for kk in range(LANES):
                    r = i0 + kk
                    pr = prow[kk] + r
                    sls = [pl.ds(j * LANES, LANES) for j in range(EV)]
                    ts = [rows[k][r, sl] for sl in sls]
                    ps = [posseg_v[pr, sl] for sl in sls]
                    for j in range(EV):
                        wb[k][r, sls[j]] = ts[j] + ps[j]
                return 0

            lax.fori_loop(0, CL // LANES, group_body, 0)
            pltpu.async_copy(wb[k], out_hbm.at[pl.ds(base, CL)], osem[k])

        fetch_and_gather(0, 0)

        def pair_body(t, _):
            for k in (0, 1):
                c = t * 2 + k

                @pl.when(c + 1 < BPW)
                def _():
                    fetch_and_gather(c + 1, k ^ 1)

                compute_and_write(c, k)
            return 0

        lax.fori_loop(0, BPW // 2, pair_body, 0)

        for c in (BPW - 2, BPW - 1):
            k = c % 2
            base = pl.multiple_of(wbase + lc * CL + c * L, CL)
            pltpu.make_async_copy(wb[k], out_hbm.at[pl.ds(base, CL)],
                                  osem[k]).wait()


@functools.partial(jax.jit, static_argnames=())
def kernel(sequence, segment_label, token_table, position_table,
           segment_table):
    seq = sequence.astype(jnp.int32)
    seg = segment_label.astype(jnp.int32)

    mesh = plsc.VectorSubcoreMesh(core_axis_name="c", subcore_axis_name="s",
                                  num_cores=NC, num_subcores=NS)
    out = pl.kernel(
        _emb_body,
        out_type=jax.ShapeDtypeStruct((B * L, E), jnp.float32),
        mesh=mesh,
        scratch_types=[
            pltpu.VMEM((BPW, CL), jnp.int32),
            pltpu.VMEM((BPW, CL), jnp.int32),
            pltpu.VMEM((CL, E), jnp.float32),
            pltpu.VMEM((CL, E), jnp.float32),
            pltpu.VMEM((CL, E), jnp.float32),
            pltpu.VMEM((CL, E), jnp.float32),
            pltpu.VMEM((3 * CL, E), jnp.float32),
            pltpu.VMEM((3, E), jnp.float32),
            pltpu.SemaphoreType.DMA,
            pltpu.SemaphoreType.DMA,
            pltpu.SemaphoreType.DMA,
            pltpu.SemaphoreType.DMA,
        ],
    )(seq, seg, token_table, position_table, segment_table)
    return out.reshape(B, L, E)
